# Initial kernel scaffold; baseline (speedup 1.0000x reference)
import jax
import jax.numpy as jnp
from jax.experimental import pallas as pl


def kernel(tw_start, pe):
    raise NotImplementedError("write your pallas kernel here")


# speedup vs baseline: 4.6801x; 4.6801x over previous
import functools

import jax
import jax.numpy as jnp
from jax import lax
from jax.experimental import pallas as pl
from jax.experimental.pallas import tpu as pltpu
from jax.experimental.pallas import tpu_sc as plsc

B = 16384
N = 200
D = 64
MAXP = 200


RBL = 512


def _rank_body(twt_ref, rankt_ref):
    x = twt_ref[...]
    iot = lax.broadcasted_iota(jnp.int32, (N, RBL), 0)

    def gbody(g, acc):
        base = pl.multiple_of(g * 8, 8)
        vg = twt_ref[pl.ds(base, 8), :]
        for s in range(8):
            vk = vg[s:s + 1, :]
            lt = vk < x
            le = vk <= x
            c = le & (lt | (iot > g * 8 + s))
            acc = jnp.where(c, acc + 1, acc)
        return acc

    acc = lax.fori_loop(0, N // 8, gbody, jnp.zeros((N, RBL), jnp.int32))
    rankt_ref[...] = acc


def _ranks_tc(twt):
    return pl.pallas_call(
        _rank_body,
        out_shape=jax.ShapeDtypeStruct((N, B), jnp.int32),
        grid=(B // RBL,),
        in_specs=[pl.BlockSpec((N, RBL), lambda i: (0, i))],
        out_specs=pl.BlockSpec((N, RBL), lambda i: (0, i)),
    )(twt)


NC = 2
NS = 16
NW = NC * NS
TOTAL = B * N
PER_W = TOTAL // NW
CHUNK = 128
STEPS = PER_W // CHUNK


def _sc_gather_body(pe_hbm, idx_hbm, out_hbm, idx_v, rows_v, sem):
    wid = lax.axis_index("s") * NC + lax.axis_index("c")
    base = wid * PER_W

    def step(i, carry):
        off = base + i * CHUNK
        pltpu.sync_copy(idx_hbm.at[pl.ds(off, CHUNK)], idx_v)
        pltpu.async_copy(pe_hbm.at[idx_v], rows_v, sem).wait()
        pltpu.sync_copy(rows_v, out_hbm.at[pl.ds(off, CHUNK)])
        return carry

    lax.fori_loop(0, STEPS, step, 0)


_sc_gather = functools.partial(
    pl.kernel,
    out_type=jax.ShapeDtypeStruct((TOTAL, D), jnp.float32),
    mesh=plsc.VectorSubcoreMesh(
        core_axis_name="c", subcore_axis_name="s", num_cores=NC,
        num_subcores=NS),
    scratch_types=[
        pltpu.VMEM((CHUNK,), jnp.int32),
        pltpu.VMEM((CHUNK, D), jnp.float32),
        pltpu.SemaphoreType.DMA,
    ],
    compiler_params=pltpu.CompilerParams(use_tc_tiling_on_sc=False),
)(_sc_gather_body)


def kernel(tw_start, pe):
    rank_t = _ranks_tc(tw_start.T)
    out = _sc_gather(pe, rank_t.T.reshape(TOTAL))
    return out.reshape(B, N, D)


# speedup vs baseline: 5.7713x; 1.2331x over previous
import functools

import jax
import jax.numpy as jnp
from jax import lax
from jax.experimental import pallas as pl
from jax.experimental.pallas import tpu as pltpu
from jax.experimental.pallas import tpu_sc as plsc

B = 16384
N = 200
D = 64
MAXP = 200


RBL = 512
NG = N // 8


def _sortable_keys(x):
    u = lax.bitcast_convert_type(x, jnp.int32)
    sgn = lax.shift_right_arithmetic(u, 31)
    return u ^ lax.shift_right_logical(sgn, 1)


def _rank_body(twt_ref, rank_ref, keys_ref, acc_ref):
    keys_ref[...] = _sortable_keys(twt_ref[...])
    acc_ref[...] = jnp.zeros((N, RBL), jnp.int32)

    iot_loc = lax.broadcasted_iota(jnp.int32, (8, RBL), 0)

    def gbody(g, _):
        base = pl.multiple_of(g * 8, 8)
        vg = keys_ref[pl.ds(base, 8), :]
        vkb = [jnp.broadcast_to(vg[s:s + 1, :], (8, RBL)) for s in range(8)]
        for jt in range(NG):
            tj = keys_ref[jt * 8:(jt + 1) * 8, :]
            thr = tj + jnp.where(jt > g, 1, 0).astype(jnp.int32)
            acc = acc_ref[jt * 8:(jt + 1) * 8, :]
            for s in range(8):
                c = vkb[s] < thr
                acc = jnp.where(c, acc + 1, acc)
            acc_ref[jt * 8:(jt + 1) * 8, :] = acc
        accd = acc_ref[pl.ds(base, 8), :]
        for s in range(8):
            m = (vkb[s] == vg) & (iot_loc > s)
            accd = jnp.where(m, accd + 1, accd)
        acc_ref[pl.ds(base, 8), :] = accd
        return 0

    lax.fori_loop(0, NG, gbody, 0)
    rank_ref[...] = jnp.transpose(acc_ref[...], (1, 0))


def _ranks_tc(twt):
    return pl.pallas_call(
        _rank_body,
        out_shape=jax.ShapeDtypeStruct((B, N), jnp.int32),
        grid=(B // RBL,),
        in_specs=[pl.BlockSpec((N, RBL), lambda i: (0, i))],
        out_specs=pl.BlockSpec((RBL, N), lambda i: (i, 0)),
        scratch_shapes=[
            pltpu.VMEM((N, RBL), jnp.int32),
            pltpu.VMEM((N, RBL), jnp.int32),
        ],
    )(twt)


NC = 2
NS = 16
NW = NC * NS
TOTAL = B * N
PER_W = TOTAL // NW
CHUNK = 128
GPS = 4
SUPER = CHUNK * GPS
STEPS = PER_W // SUPER
NBUF = 2


def _sc_gather_body(pe_hbm, idx_hbm, out_hbm, idx_v, rows_v, gsem, wsem):
    wid = lax.axis_index("s") * NC + lax.axis_index("c")
    base = wid * PER_W
    irow0 = wid * (PER_W // CHUNK)

    def step(it, carry):
        for buf in range(NBUF):
            i = it * NBUF + buf
            off = base + i * SUPER

            @pl.when(it >= 1)
            def _():
                pltpu.make_async_copy(
                    rows_v.at[buf], out_hbm.at[pl.ds(off, SUPER)],
                    wsem[buf]).wait()

            pltpu.sync_copy(idx_hbm.at[pl.ds(irow0 + i * GPS, GPS)],
                            idx_v.at[buf])
            copies = []
            for q in range(GPS):
                copies.append(pltpu.async_copy(
                    pe_hbm.at[idx_v.at[buf, q]],
                    rows_v.at[buf, pl.ds(q * CHUNK, CHUNK)],
                    gsem[buf]))
            for c in copies:
                c.wait()
            pltpu.async_copy(rows_v.at[buf],
                             out_hbm.at[pl.ds(off, SUPER)], wsem[buf])
        return carry

    lax.fori_loop(0, STEPS // NBUF, step, 0)

    for buf in range(NBUF):
        i_last = STEPS - NBUF + buf
        pltpu.make_async_copy(
            rows_v.at[buf],
            out_hbm.at[pl.ds(base + i_last * SUPER, SUPER)],
            wsem[buf]).wait()


_sc_gather = functools.partial(
    pl.kernel,
    out_type=jax.ShapeDtypeStruct((TOTAL, D), jnp.float32),
    mesh=plsc.VectorSubcoreMesh(
        core_axis_name="c", subcore_axis_name="s", num_cores=NC,
        num_subcores=NS),
    scratch_types=[
        pltpu.VMEM((NBUF, GPS, CHUNK), jnp.int32),
        pltpu.VMEM((NBUF, SUPER, D), jnp.float32),
        [pltpu.SemaphoreType.DMA] * NBUF,
        [pltpu.SemaphoreType.DMA] * NBUF,
    ],
    compiler_params=pltpu.CompilerParams(use_tc_tiling_on_sc=False),
)(_sc_gather_body)


def kernel(tw_start, pe):
    rank = _ranks_tc(tw_start.T)
    out = _sc_gather(pe, rank.reshape(TOTAL // CHUNK, CHUNK))
    return out.reshape(B, N, D)


# speedup vs baseline: 9.5727x; 1.6587x over previous
import functools

import jax
import jax.numpy as jnp
from jax import lax
from jax.experimental import pallas as pl
from jax.experimental.pallas import tpu as pltpu
from jax.experimental.pallas import tpu_sc as plsc

B = 16384
N = 200
D = 64
MAXP = 200


RBL = 512
NG = N // 8


def _sortable_keys(x):
    u = lax.bitcast_convert_type(x, jnp.int32)
    sgn = lax.shift_right_arithmetic(u, 31)
    return u ^ lax.shift_right_logical(sgn, 1)


def _rank_body(tw_ref, rank_ref, keys_ref, acc_ref):
    keys_ref[...] = _sortable_keys(jnp.transpose(tw_ref[...], (1, 0)))
    acc_ref[...] = jnp.zeros((N, RBL), jnp.int32)

    iot_loc = lax.broadcasted_iota(jnp.int32, (8, RBL), 0)

    def gbody(g, _):
        base = pl.multiple_of(g * 8, 8)
        vg = keys_ref[pl.ds(base, 8), :]
        vkb = [jnp.broadcast_to(vg[s:s + 1, :], (8, RBL)) for s in range(8)]
        for jt in range(NG):
            tj = keys_ref[jt * 8:(jt + 1) * 8, :]
            thr = tj + jnp.where(jt > g, 1, 0).astype(jnp.int32)
            acc = acc_ref[jt * 8:(jt + 1) * 8, :]
            for s in range(8):
                c = vkb[s] < thr
                acc = jnp.where(c, acc + 1, acc)
            acc_ref[jt * 8:(jt + 1) * 8, :] = acc
        accd = acc_ref[pl.ds(base, 8), :]
        for s in range(8):
            m = (vkb[s] == vg) & (iot_loc > s)
            accd = jnp.where(m, accd + 1, accd)
        acc_ref[pl.ds(base, 8), :] = accd
        return 0

    lax.fori_loop(0, NG, gbody, 0)
    rank_ref[...] = jnp.transpose(acc_ref[...], (1, 0))


def _ranks_tc(tw):
    return pl.pallas_call(
        _rank_body,
        out_shape=jax.ShapeDtypeStruct((B, N), jnp.int32),
        grid=(B // RBL,),
        in_specs=[pl.BlockSpec((RBL, N), lambda i: (i, 0))],
        out_specs=pl.BlockSpec((RBL, N), lambda i: (i, 0)),
        scratch_shapes=[
            pltpu.VMEM((N, RBL), jnp.int32),
            pltpu.VMEM((N, RBL), jnp.int32),
        ],
    )(tw)


NC = 2
NS = 16
NW = NC * NS
TOTAL = B * N
PER_W = TOTAL // NW
CHUNK = 128
GPS = 4
SUPER = CHUNK * GPS
STEPS = PER_W // SUPER
NBUF = 2


def _sc_gather_body(pe_hbm, idx_hbm, out_hbm, pe_v, idx_v, rows_v, gsem, wsem):
    wid = lax.axis_index("s") * NC + lax.axis_index("c")
    base = wid * PER_W
    irow0 = wid * (PER_W // CHUNK)

    @pl.when(lax.axis_index("s") == 0)
    def _():
        pltpu.sync_copy(pe_hbm, pe_v)
    plsc.subcore_barrier()

    def step(it, carry):
        for buf in range(NBUF):
            i = it * NBUF + buf
            off = base + i * SUPER

            @pl.when(it >= 1)
            def _():
                pltpu.make_async_copy(
                    rows_v.at[buf], out_hbm.at[pl.ds(off, SUPER)],
                    wsem[buf]).wait()

            pltpu.sync_copy(idx_hbm.at[pl.ds(irow0 + i * GPS, GPS)],
                            idx_v.at[buf])
            copies = []
            for q in range(GPS):
                copies.append(pltpu.async_copy(
                    pe_v.at[idx_v.at[buf, q]],
                    rows_v.at[buf, pl.ds(q * CHUNK, CHUNK)],
                    gsem[buf]))
            for c in copies:
                c.wait()
            pltpu.async_copy(rows_v.at[buf],
                             out_hbm.at[pl.ds(off, SUPER)], wsem[buf])
        return carry

    lax.fori_loop(0, STEPS // NBUF, step, 0)

    for buf in range(NBUF):
        i_last = STEPS - NBUF + buf
        pltpu.make_async_copy(
            rows_v.at[buf],
            out_hbm.at[pl.ds(base + i_last * SUPER, SUPER)],
            wsem[buf]).wait()


_sc_gather = functools.partial(
    pl.kernel,
    out_type=jax.ShapeDtypeStruct((TOTAL, D), jnp.float32),
    mesh=plsc.VectorSubcoreMesh(
        core_axis_name="c", subcore_axis_name="s", num_cores=NC,
        num_subcores=NS),
    scratch_types=[
        pltpu.VMEM_SHARED((MAXP, D), jnp.float32),
        pltpu.VMEM((NBUF, GPS, CHUNK), jnp.int32),
        pltpu.VMEM((NBUF, SUPER, D), jnp.float32),
        [pltpu.SemaphoreType.DMA] * NBUF,
        [pltpu.SemaphoreType.DMA] * NBUF,
    ],
    compiler_params=pltpu.CompilerParams(use_tc_tiling_on_sc=False),
)(_sc_gather_body)


def kernel(tw_start, pe):
    rank = _ranks_tc(tw_start)
    out = _sc_gather(pe, rank.reshape(TOTAL // CHUNK, CHUNK))
    return out.reshape(B, N, D)
